# baseline (speedup 1.0000x reference)
import functools

import jax
import jax.numpy as jnp
from jax.experimental import pallas as pl
from jax.experimental.pallas import tpu as pltpu


def _head_kernel(x_ref, w_ref, b_ref, o_ref, *, inv_hw):
    hw = x_ref.shape[0]
    acc = x_ref[0]
    for h in range(1, hw):
        acc = acc + x_ref[h]
    scores = jax.lax.dot_general(
        acc,
        w_ref[...],
        dimension_numbers=(((1,), (1,)), ((), ())),
        preferred_element_type=jnp.float32,
    )
    o_ref[...] = scores * inv_hw + b_ref[...]


def _largest_divisor_at_most(n, cap):
    for d in range(min(n, cap), 0, -1):
        if n % d == 0:
            return d
    return 1


def kernel(x_nchw, weight, bias):
    B, C, H, W = x_nchw.shape
    N = weight.shape[0]
    HW = H * W
    out_dtype = jnp.result_type(x_nchw.dtype, weight.dtype)

    xp = x_nchw.transpose(2, 3, 0, 1).reshape(HW, B, C)
    bias2 = bias.reshape(1, N)

    bt = _largest_divisor_at_most(B, 16)
    grid = (B // bt,)

    cost = pl.CostEstimate(
        flops=B * C * HW + 2 * B * C * N,
        transcendentals=0,
        bytes_accessed=xp.size * 4 + C * N * 4 + B * N * 4,
    )

    scores = pl.pallas_call(
        functools.partial(_head_kernel, inv_hw=float(1.0 / HW)),
        out_shape=jax.ShapeDtypeStruct((B, N), jnp.float32),
        grid=grid,
        in_specs=[
            pl.BlockSpec((HW, bt, C), lambda i: (0, i, 0)),
            pl.BlockSpec((N, C), lambda i: (0, 0)),
            pl.BlockSpec((1, N), lambda i: (0, 0)),
        ],
        out_specs=pl.BlockSpec((bt, N), lambda i: (i, 0)),
        compiler_params=pltpu.CompilerParams(
            dimension_semantics=("parallel",),
            vmem_limit_bytes=48 << 20,
        ),
        cost_estimate=cost,
    )(xp, weight, bias2)

    return scores.astype(out_dtype)


# speedup vs baseline: 1.0029x; 1.0029x over previous
import functools

import jax
import jax.numpy as jnp
from jax.experimental import pallas as pl
from jax.experimental.pallas import tpu as pltpu


def _head_kernel(x_ref, w_ref, b_ref, o_ref, *, inv_hw):
    hw = x_ref.shape[0]
    acc = x_ref[0]
    for h in range(1, hw):
        acc = acc + x_ref[h]
    scores = jax.lax.dot_general(
        acc,
        w_ref[...],
        dimension_numbers=(((1,), (1,)), ((), ())),
        preferred_element_type=jnp.float32,
    )
    o_ref[...] = scores * inv_hw + b_ref[...]


def _largest_divisor_at_most(n, cap):
    for d in range(min(n, cap), 0, -1):
        if n % d == 0:
            return d
    return 1


def kernel(x_nchw, weight, bias):
    B, C, H, W = x_nchw.shape
    N = weight.shape[0]
    HW = H * W
    out_dtype = jnp.result_type(x_nchw.dtype, weight.dtype)

    xp = x_nchw.transpose(2, 3, 0, 1).reshape(HW, B, C)
    bias2 = bias.reshape(1, N)

    bt = _largest_divisor_at_most(B, 64)
    grid = (B // bt,)

    cost = pl.CostEstimate(
        flops=B * C * HW + 2 * B * C * N,
        transcendentals=0,
        bytes_accessed=xp.size * 4 + C * N * 4 + B * N * 4,
    )

    scores = pl.pallas_call(
        functools.partial(_head_kernel, inv_hw=float(1.0 / HW)),
        out_shape=jax.ShapeDtypeStruct((B, N), jnp.float32),
        grid=grid,
        in_specs=[
            pl.BlockSpec((HW, bt, C), lambda i: (0, i, 0)),
            pl.BlockSpec((N, C), lambda i: (0, 0)),
            pl.BlockSpec((1, N), lambda i: (0, 0)),
        ],
        out_specs=pl.BlockSpec((bt, N), lambda i: (i, 0)),
        compiler_params=pltpu.CompilerParams(
            dimension_semantics=("parallel",),
            vmem_limit_bytes=62 << 20,
        ),
        cost_estimate=cost,
    )(xp, weight, bias2)

    return scores.astype(out_dtype)


# speedup vs baseline: 1.1117x; 1.1085x over previous
import functools

import jax
import jax.numpy as jnp
from jax.experimental import pallas as pl
from jax.experimental.pallas import tpu as pltpu


def _head_kernel(x_ref, w_ref, b_ref, o_ref, *, inv_hw):
    hw = x_ref.shape[0]
    acc = x_ref[0]
    for h in range(1, hw):
        acc = acc + x_ref[h]
    scores = jax.lax.dot_general(
        acc,
        w_ref[...],
        dimension_numbers=(((1,), (1,)), ((), ())),
        preferred_element_type=jnp.float32,
    )
    o_ref[...] = scores * inv_hw + b_ref[...]


def _largest_divisor_at_most(n, cap):
    for d in range(min(n, cap), 0, -1):
        if n % d == 0:
            return d
    return 1


def kernel(x_nchw, weight, bias):
    B, C, H, W = x_nchw.shape
    N = weight.shape[0]
    HW = H * W
    out_dtype = jnp.result_type(x_nchw.dtype, weight.dtype)

    xp = x_nchw.transpose(2, 3, 0, 1).reshape(HW, B, C)
    bias2 = bias.reshape(1, N)

    bt = _largest_divisor_at_most(B, 32)
    grid = (B // bt,)

    cost = pl.CostEstimate(
        flops=B * C * HW + 2 * B * C * N,
        transcendentals=0,
        bytes_accessed=xp.size * 4 + C * N * 4 + B * N * 4,
    )

    scores = pl.pallas_call(
        functools.partial(_head_kernel, inv_hw=float(1.0 / HW)),
        out_shape=jax.ShapeDtypeStruct((B, N), jnp.float32),
        grid=grid,
        in_specs=[
            pl.BlockSpec((HW, bt, C), lambda i: (0, i, 0)),
            pl.BlockSpec((N, C), lambda i: (0, 0)),
            pl.BlockSpec((1, N), lambda i: (0, 0)),
        ],
        out_specs=pl.BlockSpec((bt, N), lambda i: (i, 0)),
        compiler_params=pltpu.CompilerParams(
            dimension_semantics=("parallel",),
            vmem_limit_bytes=48 << 20,
        ),
        cost_estimate=cost,
    )(xp, weight, bias2)

    return scores.astype(out_dtype)
